# Initial kernel scaffold; baseline (speedup 1.0000x reference)
import jax
import jax.numpy as jnp
from jax.experimental import pallas as pl


def kernel(x, spiral_indices, trans_row, trans_col, trans_val, W, b):
    raise NotImplementedError("write your pallas kernel here")


# speedup vs baseline: 1.3445x; 1.3445x over previous
import jax
import jax.numpy as jnp
from jax import lax
from jax.experimental import pallas as pl
from jax.experimental.pallas import tpu as pltpu
from jax.experimental.pallas import tpu_sc as plsc

B = 4
N = 10000
SEQ = 16
IN = 256
OUT = 256
M = 2500
NNZ = 20000

R = B * N
NW = 32
ROWS_PER_W = R // NW

_MM_RB = 400


def _mm_body(x_ref, w_ref, z_ref):
    z_ref[...] = lax.dot_general(
        x_ref[...], w_ref[...], (((1,), (0,)), ((), ())),
        preferred_element_type=jnp.float32)


def _spiral_matmul(x2, w):
    return pl.pallas_call(
        _mm_body,
        grid=(R // _MM_RB, SEQ),
        in_specs=[
            pl.BlockSpec((_MM_RB, IN), lambda i, s: (i, 0)),
            pl.BlockSpec((IN, OUT), lambda i, s: (s, 0)),
        ],
        out_specs=pl.BlockSpec((_MM_RB, OUT), lambda i, s: (i, s)),
        out_shape=jax.ShapeDtypeStruct((R, SEQ * OUT), jnp.float32),
    )(x2, w)


_GS_C = 5
_GS_NCHUNK = ROWS_PER_W // _GS_C

_SC_MESH = plsc.VectorSubcoreMesh(core_axis_name="c", subcore_axis_name="s")


def _gather_sum_body(z3_hbm, idx_hbm, bias_hbm, out_hbm,
                     myidx_v, gidx_v, grows_v, out_v, bias_v, sem):
    wid = lax.axis_index("c") * 16 + lax.axis_index("s")
    b = wid // 8
    n_base = (wid % 8) * ROWS_PER_W
    row_base0 = wid * ROWS_PER_W
    boff = b * (N * SEQ)

    pltpu.sync_copy(idx_hbm.at[pl.ds(n_base * SEQ, ROWS_PER_W * SEQ)], myidx_v)
    pltpu.sync_copy(bias_hbm, bias_v)
    iota16 = lax.broadcasted_iota(jnp.int32, (16,), 0)

    def chunk_body(c, _):
        node_off = c * _GS_C
        for j in range(_GS_C):
            idxrow = myidx_v[pl.ds((node_off + j) * SEQ, SEQ)]
            gidx_v[pl.ds(j * SEQ, SEQ)] = idxrow * SEQ + iota16 + boff
        pltpu.async_copy(z3_hbm.at[gidx_v], grows_v, sem).wait()
        for j in range(_GS_C):
            r0 = j * SEQ
            for t in range(OUT // 16):
                cs = pl.ds(t * 16, 16)
                acc = grows_v[r0, cs]
                for s in range(1, SEQ):
                    acc = acc + grows_v[r0 + s, cs]
                acc = acc + bias_v[cs]
                out_v[pl.ds(j * OUT + t * 16, 16)] = jnp.where(
                    acc > 0.0, acc, jnp.exp(acc) - 1.0)
        pltpu.sync_copy(
            out_v, out_hbm.at[pl.ds((row_base0 + node_off) * OUT, _GS_C * OUT)])
        return 0

    lax.fori_loop(0, _GS_NCHUNK, chunk_body, 0)


def _gather_sum(z3, spiral_idx, bias):
    f = pl.kernel(
        _gather_sum_body,
        out_type=jax.ShapeDtypeStruct((R * OUT,), jnp.float32),
        mesh=_SC_MESH,
        scratch_types=[
            pltpu.VMEM((ROWS_PER_W * SEQ,), jnp.int32),
            pltpu.VMEM((_GS_C * SEQ,), jnp.int32),
            pltpu.VMEM((_GS_C * SEQ, OUT), jnp.float32),
            pltpu.VMEM((_GS_C * OUT,), jnp.float32),
            pltpu.VMEM((OUT,), jnp.float32),
            pltpu.SemaphoreType.DMA,
        ],
    )
    return f(z3, spiral_idx, bias)


NNZ_PAD = 20480
_P_C = 128
M_PAD = 2504
_RPT = 312
_RPT_LAST = M_PAD - 7 * _RPT
_BCAST_DNUMS = lax.GatherDimensionNumbers(
    offset_dims=(), collapsed_slice_dims=(0,), start_index_map=(0,))


def _lane_bcast(v, j):
    return lax.gather(v, jnp.full((16, 1), j, jnp.int32), _BCAST_DNUMS,
                      (1,), mode=lax.GatherScatterMode.PROMISE_IN_BOUNDS)


def _pool_body(out2_hbm, rows_hbm, cols_hbm, vals_hbm, bounds_hbm, pooled_hbm,
               rows_v, cols_v, vals_v, gidx_v, gbuf_v, acc_v, bounds_v, sem):
    cid = lax.axis_index("c")
    sid = lax.axis_index("s")
    wid = cid * 16 + sid
    b = 2 * cid + sid // 8
    t8 = sid % 8
    r_lo = t8 * _RPT
    iota16 = lax.broadcasted_iota(jnp.int32, (16,), 0)

    def zrow(i, _):
        for t in range(OUT // 16):
            acc_v[i, pl.ds(t * 16, 16)] = jnp.zeros((16,), jnp.float32)
        return 0
    lax.fori_loop(0, _RPT_LAST, zrow, 0)

    pltpu.sync_copy(bounds_hbm.at[pl.ds(wid * 16, 16)], bounds_v)
    bvec = bounds_v[pl.ds(0, 16)]
    ent_lo = bvec[0]
    ent_hi = bvec[1]
    ent_al = ent_lo & ~(_P_C - 1)
    nch = (ent_hi - ent_al + _P_C - 1) // _P_C

    def chunk_body(c, _):
        e_base = pl.multiple_of(ent_al + c * _P_C, _P_C)
        pltpu.sync_copy(rows_hbm.at[pl.ds(e_base, _P_C)], rows_v)
        pltpu.sync_copy(cols_hbm.at[pl.ds(e_base, _P_C)], cols_v)
        pltpu.sync_copy(vals_hbm.at[pl.ds(e_base, _P_C)], vals_v)
        for k in range(_P_C // 16):
            ks = pl.ds(k * 16, 16)
            gidx_v[ks] = cols_v[ks] + b * N
        pltpu.async_copy(out2_hbm.at[gidx_v], gbuf_v, sem).wait()

        def group_body(g, _):
            gs = pl.ds(g * 16, 16)
            eid = e_base + g * 16 + iota16
            valid = (eid >= ent_lo) & (eid < ent_hi)
            bcv = jnp.where(valid, vals_v[gs], 0.0)
            rowsv = rows_v[gs]
            for j in range(16):
                bc = _lane_bcast(bcv, j)
                lrow = jnp.clip(rowsv[j] - r_lo, 0, _RPT_LAST - 1)
                e = g * 16 + j
                for t in range(OUT // 16):
                    cs = pl.ds(t * 16, 16)
                    acc_v[lrow, cs] = acc_v[lrow, cs] + gbuf_v[e, cs] * bc
            return 0
        lax.fori_loop(0, _P_C // 16, group_body, 0)
        return 0

    lax.fori_loop(0, nch, chunk_body, 0)

    @pl.when(t8 < 7)
    def _():
        pltpu.sync_copy(acc_v.at[pl.ds(0, _RPT)],
                        pooled_hbm.at[b].at[pl.ds(r_lo, _RPT)])

    @pl.when(t8 == 7)
    def _():
        pltpu.sync_copy(acc_v, pooled_hbm.at[b].at[pl.ds(7 * _RPT, _RPT_LAST)])


def _pool(out2, rows_p, cols_p, vals_p, bounds):
    f = pl.kernel(
        _pool_body,
        out_type=jax.ShapeDtypeStruct((B, M_PAD, OUT), jnp.float32),
        mesh=_SC_MESH,
        scratch_types=[
            pltpu.VMEM((_P_C,), jnp.int32),
            pltpu.VMEM((_P_C,), jnp.int32),
            pltpu.VMEM((_P_C,), jnp.float32),
            pltpu.VMEM((_P_C,), jnp.int32),
            pltpu.VMEM((_P_C, OUT), jnp.float32),
            pltpu.VMEM((_RPT_LAST, OUT), jnp.float32),
            pltpu.VMEM((16,), jnp.int32),
            pltpu.SemaphoreType.DMA,
        ],
    )
    return f(out2, rows_p, cols_p, vals_p, bounds)


def kernel(x, spiral_indices, trans_row, trans_col, trans_val, W, b):
    x2 = x.reshape(R, IN)
    z = _spiral_matmul(x2, W)
    z3 = z.reshape(R * SEQ, OUT)
    idx_flat = spiral_indices.astype(jnp.int32).reshape(N * SEQ)
    out2 = _gather_sum(z3, idx_flat, b).reshape(R, OUT)

    pad = NNZ_PAD - NNZ
    rows32 = trans_row.astype(jnp.int32)
    rows_p = jnp.concatenate([rows32, jnp.zeros((pad,), jnp.int32)])
    cols_p = jnp.concatenate(
        [trans_col.astype(jnp.int32), jnp.zeros((pad,), jnp.int32)])
    vals_p = jnp.concatenate([trans_val, jnp.zeros((pad,), jnp.float32)])
    starts = jnp.arange(8, dtype=jnp.int32) * _RPT
    ss = jnp.searchsorted(rows32, starts, side="left").astype(jnp.int32)
    ends = jnp.concatenate([ss[1:], jnp.array([NNZ], jnp.int32)])
    pair16 = jnp.pad(jnp.stack([ss, ends], axis=1), ((0, 0), (0, 14)))
    bounds = jnp.tile(pair16, (4, 1)).reshape(32 * 16)
    return _pool(out2, rows_p, cols_p, vals_p, bounds)[:, :M, :]


# speedup vs baseline: 2.1426x; 1.5936x over previous
import jax
import jax.numpy as jnp
from jax import lax
from jax.experimental import pallas as pl
from jax.experimental.pallas import tpu as pltpu
from jax.experimental.pallas import tpu_sc as plsc

B = 4
N = 10000
SEQ = 16
IN = 256
OUT = 256
M = 2500
NNZ = 20000

R = B * N
NW = 32
ROWS_PER_W = R // NW

_MM_RB = 400


def _mm_body(x_ref, w_ref, z_ref):
    z_ref[...] = lax.dot_general(
        x_ref[...], w_ref[...], (((1,), (0,)), ((), ())),
        preferred_element_type=jnp.float32)


def _spiral_matmul(x2, w):
    nib = R // _MM_RB
    return pl.pallas_call(
        _mm_body,
        grid=(nib, SEQ),
        in_specs=[
            pl.BlockSpec((_MM_RB, IN), lambda i, s: (i, 0)),
            pl.BlockSpec((IN, OUT), lambda i, s: (s, 0)),
        ],
        out_specs=pl.BlockSpec((_MM_RB, OUT), lambda i, s: (s * nib + i, 0)),
        out_shape=jax.ShapeDtypeStruct((SEQ * R, OUT), jnp.float32),
    )(x2, w)


_GS_CH = 8
_GS_TR = 1248
_GS_MAXCH = 164
_GS_NIB = 11

_SC_MESH = plsc.VectorSubcoreMesh(core_axis_name="c", subcore_axis_name="s")


def _gather_sum_body(z4_hbm, idx_hbm, bias_hbm, out_hbm,
                     idx8_v, gidx_a, gidx_b, grow_a, grow_b,
                     outb_v, bias_v, sem_a, sem_b):
    wid = lax.axis_index("c") * 16 + lax.axis_index("s")
    row_base = wid * _GS_TR
    nrows = jnp.where(wid == 31, R - 31 * _GS_TR, _GS_TR)
    nch = nrows // _GS_CH
    pltpu.sync_copy(bias_hbm, bias_v)
    iota16 = lax.broadcasted_iota(jnp.int32, (16,), 0)

    def build_start(k, gidx, grow, sem):
        r0 = row_base + k * _GS_CH
        n0 = pl.multiple_of(r0 % N, _GS_CH)
        pltpu.sync_copy(idx_hbm.at[pl.ds(n0, _GS_CH)], idx8_v)
        svec = iota16 * R + (r0 // N) * N
        for j in range(_GS_CH):
            gidx[pl.ds(j * SEQ, SEQ)] = idx8_v[j, :] + svec
        return pltpu.async_copy(z4_hbm.at[gidx], grow, sem)

    def wait_for(gidx, grow, sem):
        pltpu.make_async_copy(z4_hbm.at[gidx], grow, sem).wait()

    def compute(k, grow):
        def node(j, _):
            for t in range(OUT // 16):
                cs = pl.ds(t * 16, 16)
                acc = grow[j * SEQ, cs]
                for s in range(1, SEQ):
                    acc = acc + grow[j * SEQ + s, cs]
                acc = acc + bias_v[cs]
                outb_v[j, cs] = jnp.where(acc > 0.0, acc, jnp.exp(acc) - 1.0)
            return 0
        lax.fori_loop(0, _GS_CH, node, 0)
        r0 = pl.multiple_of(row_base + k * _GS_CH, _GS_CH)
        pltpu.sync_copy(outb_v, out_hbm.at[pl.ds(r0, _GS_CH)])

    build_start(0, gidx_a, grow_a, sem_a)

    def pair(k2, _):
        k = 2 * k2
        build_start(k + 1, gidx_b, grow_b, sem_b)
        wait_for(gidx_a, grow_a, sem_a)
        compute(k, grow_a)

        @pl.when(k + 2 < nch)
        def _():
            build_start(k + 2, gidx_a, grow_a, sem_a)
        wait_for(gidx_b, grow_b, sem_b)
        compute(k + 1, grow_b)
        return 0

    lax.fori_loop(0, nch // 2, pair, 0)


def _gather_sum(z4, spiral_idx, bias):
    f = pl.kernel(
        _gather_sum_body,
        out_type=jax.ShapeDtypeStruct((R, OUT), jnp.float32),
        mesh=_SC_MESH,
        scratch_types=[
            pltpu.VMEM((_GS_CH, SEQ), jnp.int32),
            pltpu.VMEM((_GS_CH * SEQ,), jnp.int32),
            pltpu.VMEM((_GS_CH * SEQ,), jnp.int32),
            pltpu.VMEM((_GS_CH * SEQ, OUT), jnp.float32),
            pltpu.VMEM((_GS_CH * SEQ, OUT), jnp.float32),
            pltpu.VMEM((_GS_CH, OUT), jnp.float32),
            pltpu.VMEM((OUT,), jnp.float32),
            pltpu.SemaphoreType.DMA,
            pltpu.SemaphoreType.DMA,
        ],
    )
    return f(z4, spiral_idx, bias)


NNZ_PAD = 20480
_P_C = 128
M_PAD = 2504
_RPT = 312
_RPT_LAST = M_PAD - 7 * _RPT
_BCAST_DNUMS = lax.GatherDimensionNumbers(
    offset_dims=(), collapsed_slice_dims=(0,), start_index_map=(0,))


def _lane_bcast(v, j):
    return lax.gather(v, jnp.full((16, 1), j, jnp.int32), _BCAST_DNUMS,
                      (1,), mode=lax.GatherScatterMode.PROMISE_IN_BOUNDS)


def _pool_body(out2_hbm, rows_hbm, cols_hbm, vals_hbm, bounds_hbm, pooled_hbm,
               rows_v, cols_v, vals_v, gidx_v, gbuf_v, acc_v, bounds_v, sem):
    cid = lax.axis_index("c")
    sid = lax.axis_index("s")
    wid = cid * 16 + sid
    b = 2 * cid + sid // 8
    t8 = sid % 8
    r_lo = t8 * _RPT
    iota16 = lax.broadcasted_iota(jnp.int32, (16,), 0)

    def zrow(i, _):
        for t in range(OUT // 16):
            acc_v[i, pl.ds(t * 16, 16)] = jnp.zeros((16,), jnp.float32)
        return 0
    lax.fori_loop(0, _RPT_LAST, zrow, 0)

    pltpu.sync_copy(bounds_hbm.at[pl.ds(wid * 16, 16)], bounds_v)
    bvec = bounds_v[pl.ds(0, 16)]
    ent_lo = bvec[0]
    ent_hi = bvec[1]
    ent_al = ent_lo & ~(_P_C - 1)
    nch = (ent_hi - ent_al + _P_C - 1) // _P_C

    def chunk_body(c, _):
        e_base = pl.multiple_of(ent_al + c * _P_C, _P_C)
        pltpu.sync_copy(rows_hbm.at[pl.ds(e_base, _P_C)], rows_v)
        pltpu.sync_copy(cols_hbm.at[pl.ds(e_base, _P_C)], cols_v)
        pltpu.sync_copy(vals_hbm.at[pl.ds(e_base, _P_C)], vals_v)
        for k in range(_P_C // 16):
            ks = pl.ds(k * 16, 16)
            gidx_v[ks] = cols_v[ks] + b * N
        pltpu.async_copy(out2_hbm.at[gidx_v], gbuf_v, sem).wait()

        def group_body(g, _):
            gs = pl.ds(g * 16, 16)
            eid = e_base + g * 16 + iota16
            valid = (eid >= ent_lo) & (eid < ent_hi)
            bcv = jnp.where(valid, vals_v[gs], 0.0)
            rowsv = rows_v[gs]
            for j in range(16):
                bc = _lane_bcast(bcv, j)
                lrow = jnp.clip(rowsv[j] - r_lo, 0, _RPT_LAST - 1)
                e = g * 16 + j
                for t in range(OUT // 16):
                    cs = pl.ds(t * 16, 16)
                    acc_v[lrow, cs] = acc_v[lrow, cs] + gbuf_v[e, cs] * bc
            return 0
        lax.fori_loop(0, _P_C // 16, group_body, 0)
        return 0

    lax.fori_loop(0, nch, chunk_body, 0)

    @pl.when(t8 < 7)
    def _():
        pltpu.sync_copy(acc_v.at[pl.ds(0, _RPT)],
                        pooled_hbm.at[b].at[pl.ds(r_lo, _RPT)])

    @pl.when(t8 == 7)
    def _():
        pltpu.sync_copy(acc_v, pooled_hbm.at[b].at[pl.ds(7 * _RPT, _RPT_LAST)])


def _pool(out2, rows_p, cols_p, vals_p, bounds):
    f = pl.kernel(
        _pool_body,
        out_type=jax.ShapeDtypeStruct((B, M_PAD, OUT), jnp.float32),
        mesh=_SC_MESH,
        scratch_types=[
            pltpu.VMEM((_P_C,), jnp.int32),
            pltpu.VMEM((_P_C,), jnp.int32),
            pltpu.VMEM((_P_C,), jnp.float32),
            pltpu.VMEM((_P_C,), jnp.int32),
            pltpu.VMEM((_P_C, OUT), jnp.float32),
            pltpu.VMEM((_RPT_LAST, OUT), jnp.float32),
            pltpu.VMEM((16,), jnp.int32),
            pltpu.SemaphoreType.DMA,
        ],
    )
    return f(out2, rows_p, cols_p, vals_p, bounds)


def kernel(x, spiral_indices, trans_row, trans_col, trans_val, W, b):
    x2 = x.reshape(R, IN)
    z4 = _spiral_matmul(x2, W)
    out2 = _gather_sum(z4, spiral_indices.astype(jnp.int32), b)

    pad = NNZ_PAD - NNZ
    rows32 = trans_row.astype(jnp.int32)
    rows_p = jnp.concatenate([rows32, jnp.zeros((pad,), jnp.int32)])
    cols_p = jnp.concatenate(
        [trans_col.astype(jnp.int32), jnp.zeros((pad,), jnp.int32)])
    vals_p = jnp.concatenate([trans_val, jnp.zeros((pad,), jnp.float32)])
    starts = jnp.arange(8, dtype=jnp.int32) * _RPT
    ss = jnp.searchsorted(rows32, starts, side="left").astype(jnp.int32)
    ends = jnp.concatenate([ss[1:], jnp.array([NNZ], jnp.int32)])
    pair16 = jnp.pad(jnp.stack([ss, ends], axis=1), ((0, 0), (0, 14)))
    bounds = jnp.tile(pair16, (4, 1)).reshape(32 * 16)
    return _pool(out2, rows_p, cols_p, vals_p, bounds)[:, :M, :]
